# baseline (speedup 1.0000x reference)
import functools

import jax
import jax.numpy as jnp
from jax import lax
from jax.experimental import pallas as pl
from jax.experimental.pallas import tpu as pltpu
from jax.experimental.pallas import tpu_sc as plsc

D = 64
BATCH = 16384
BLK_U = 4096
BLK_I = 1024
C = 1024
SUBC = 384
B_SENT = BATCH
GATH_ROWS = BATCH + 16


@functools.cache
def _build(n_user: int, n_item: int):
    info = plsc.get_sparse_core_info()
    nc, ns, lanes = info.num_cores, info.num_subcores, info.num_lanes
    nw = nc * ns
    b_per_w = BATCH // nw

    u_nblk = n_user // BLK_U
    u_main = u_nblk * BLK_U
    u_full = (n_user // 128) * 128
    u_epi_w = 8
    u_epi_main_w = u_full - u_main
    u_tail_w = n_user - u_full
    i_nblk = n_item // BLK_I
    i_main = i_nblk * BLK_I
    i_full = (n_item // 128) * 128
    i_epi_w = 24
    i_epi_main_w = i_full - i_main
    i_tail_w = n_item - i_full
    i_nsub = 6
    i_sub_w = -(-(n_item - i_main) // i_nsub)

    mesh = plsc.VectorSubcoreMesh(core_axis_name="c", subcore_axis_name="s")
    cparams = pltpu.CompilerParams(needs_layout_passes=False)

    @functools.partial(
        pl.kernel,
        out_type=(jax.ShapeDtypeStruct((GATH_ROWS, 128), jnp.float32),
                  jax.ShapeDtypeStruct((GATH_ROWS, 128), jnp.float32)),
        mesh=mesh,
        compiler_params=cparams,
        scratch_types=[
            pltpu.VMEM((4096,), jnp.int32),
            pltpu.VMEM((C,), jnp.int32),
            pltpu.VMEM((C,), jnp.int32),
            pltpu.VMEM((C,), jnp.int32),
            pltpu.VMEM((C,), jnp.int32),
            pltpu.VMEM((SUBC,), jnp.int32),
            pltpu.VMEM((SUBC,), jnp.int32),
            pltpu.VMEM((SUBC // 64, 64), jnp.int32),
            pltpu.VMEM((8, BLK_U), jnp.float32),
            pltpu.VMEM((8, BLK_U), jnp.float32),
            pltpu.VMEM((8, 64), jnp.float32),
            pltpu.VMEM((8, 32), jnp.float32),
            pltpu.VMEM((SUBC, 128), jnp.float32),
            pltpu.SemaphoreType.DMA,
            pltpu.SemaphoreType.DMA,
            pltpu.SemaphoreType.DMA,
        ],
    )
    def gather_kernel(user_hbm, item_hbm, utab, itab, ugath, igath,
                      idxbuf, ulist, ublist, ilist, iblist, sub, subb, subb2,
                      buf0, buf1, tailu, taili, blk_rows, sem0, sem1, ssem):
        wid = lax.axis_index("s") * nc + lax.axis_index("c")
        lane = jax.lax.iota(jnp.int32, lanes)
        bufs = (buf0, buf1)
        sems = (sem0, sem1)

        def do_filter(idx_hbm, vlist, blist, main, epi_w, shift):
            def chunk(ci, n):
                pltpu.sync_copy(idx_hbm.at[pl.ds(ci * 4096, 4096)], idxbuf)

                def inner(i, n):
                    v = idxbuf[pl.ds(i * lanes, lanes)]
                    own = jnp.where(v >= main, epi_w, (v >> shift) & 31)
                    mask = own == wid
                    plsc.store_compressed(vlist.at[pl.ds(n, lanes)], v,
                                          mask=mask)
                    bpos = ci * 4096 + i * lanes + lane
                    plsc.store_compressed(blist.at[pl.ds(n, lanes)], bpos,
                                          mask=mask)
                    cnt = lax.reduce_sum(
                        jnp.where(mask, 1, 0).astype(jnp.int32), axes=(0,))
                    return jnp.minimum(n + cnt, C - lanes)

                return lax.fori_loop(0, 4096 // lanes, inner, n)

            return lax.fori_loop(0, BATCH // 4096, chunk, jnp.int32(0))

        def clear_subb():
            for k in range(SUBC // lanes):
                subb[pl.ds(k * lanes, lanes)] = jnp.full(
                    (lanes,), B_SENT, jnp.int32)

        def refilter(vlist, blist, nloc, lo, hi):
            def f(g, m):
                v = vlist[pl.ds(g * lanes, lanes)]
                b = blist[pl.ds(g * lanes, lanes)]
                mask = (v >= lo) & (v < hi) & ((g * lanes + lane) < nloc)
                plsc.store_compressed(sub.at[pl.ds(m, lanes)], v, mask=mask)
                plsc.store_compressed(subb.at[pl.ds(m, lanes)], b, mask=mask)
                cnt = lax.reduce_sum(
                    jnp.where(mask, 1, 0).astype(jnp.int32), axes=(0,))
                return jnp.minimum(m + cnt, SUBC - lanes)

            return lax.fori_loop(0, C // lanes, f, jnp.int32(0))

        def extract_octet(src, o, cnt, col_lo, width):
            def eo(g2, _):
                @pl.when(g2 * lanes < cnt)
                def _():
                    j16 = g2 * lanes + lane
                    valid = j16 < cnt
                    v = sub[pl.ds(g2 * lanes, lanes)]
                    colc = jnp.clip(v - col_lo, 0, width - 1)
                    for dd in range(8):
                        vals = plsc.load_gather(
                            src, [jnp.full((lanes,), dd, jnp.int32), colc])
                        plsc.store_scatter(
                            blk_rows,
                            [j16, jnp.full((lanes,), o * 8 + dd, jnp.int32)],
                            vals, mask=valid)
                return 0

            lax.fori_loop(0, SUBC // lanes, eo, 0)

        def extract_octet_dual(mainbuf, tailbuf, o, cnt, col_lo, main_w,
                               tail_w):
            def eo(g2, _):
                @pl.when(g2 * lanes < cnt)
                def _():
                    j16 = g2 * lanes + lane
                    valid = j16 < cnt
                    v = sub[pl.ds(g2 * lanes, lanes)]
                    colg = v - col_lo
                    is_main = colg < main_w
                    colm = jnp.clip(colg, 0, main_w - 1)
                    colt = jnp.clip(colg - main_w, 0, tail_w - 1)
                    for dd in range(8):
                        ddv = jnp.full((lanes,), dd, jnp.int32)
                        vm = plsc.load_gather(mainbuf, [ddv, colm])
                        vt = plsc.load_gather(tailbuf, [ddv, colt])
                        plsc.store_scatter(
                            blk_rows,
                            [j16, jnp.full((lanes,), o * 8 + dd, jnp.int32)],
                            jnp.where(is_main, vm, vt), mask=valid)
                return 0

            lax.fori_loop(0, SUBC // lanes, eo, 0)

        def scatter_block(gath, cnt):
            for k in range(SUBC // 64):
                for j in range(4):
                    subb2[k, pl.ds(j * lanes, lanes)] = subb[
                        pl.ds(k * 64 + j * lanes, lanes)]
            for k in range(SUBC // 64):
                @pl.when(k * 64 < cnt)
                def _(k=k):
                    pltpu.async_copy(
                        blk_rows.at[pl.ds(k * 64, 64)],
                        gath.at[subb2.at[k]], ssem).wait()

        def main_blocks(tab, gath, vlist, blist, nloc, nblk, s2_cnt, blksz):
            def blk_body(s2, _):
                blk = wid + nw * s2

                @pl.when(blk < nblk)
                def _():
                    base = blk * blksz
                    cnt = refilter(vlist, blist, nloc, base, base + blksz)
                    dsts = (bufs[0].at[:, pl.ds(0, blksz)],
                            bufs[1].at[:, pl.ds(0, blksz)])
                    h = {0: pltpu.async_copy(
                        tab.at[pl.ds(0, 8), pl.ds(base, blksz)],
                        dsts[0], sems[0])}
                    for o in range(8):
                        if o + 1 < 8:
                            h[o + 1] = pltpu.async_copy(
                                tab.at[pl.ds((o + 1) * 8, 8),
                                       pl.ds(base, blksz)],
                                dsts[(o + 1) % 2], sems[(o + 1) % 2])
                        h[o].wait()
                        extract_octet(bufs[o % 2], o, cnt, base, blksz)
                    scatter_block(gath, cnt)

                return 0

            lax.fori_loop(0, s2_cnt, blk_body, 0)

        def epilogue(tab, gath, vlist, blist, nloc, epi_w, main_off,
                     main_w, full_off, tail_w, tailbuf, nsub, sub_w, n_rows):
            @pl.when(wid == epi_w)
            def _():
                def sub_pass(p, _):
                    lo = main_off + p * sub_w
                    hi = jnp.minimum(lo + sub_w, n_rows)
                    cnt = refilter(vlist, blist, nloc, lo, hi)
                    for o in range(8):
                        hh = []
                        for c0 in range(0, main_w, 512):
                            cw = min(512, main_w - c0)
                            hh.append(pltpu.async_copy(
                                tab.at[pl.ds(o * 8, 8),
                                       pl.ds(main_off + c0, cw)],
                                buf0.at[:, pl.ds(c0, cw)], sem0))
                        hh.append(pltpu.async_copy(
                            tab.at[pl.ds(o * 8, 8),
                                   pl.ds(full_off, tail_w)],
                            tailbuf, sem1))
                        for h in hh:
                            h.wait()
                        extract_octet_dual(buf0, tailbuf, o, cnt,
                                           main_off, main_w, tail_w)
                    scatter_block(gath, cnt)
                    return 0

                lax.fori_loop(0, nsub, sub_pass, 0)

        with jax.named_scope("filter"):
            n_u = do_filter(user_hbm, ulist, ublist, u_main, u_epi_w, 12)
            n_i = do_filter(item_hbm, ilist, iblist, i_main, i_epi_w, 10)

        clear_subb()
        with jax.named_scope("umain"):
            main_blocks(utab, ugath, ulist, ublist, n_u, u_nblk, 8, BLK_U)
        with jax.named_scope("uepi"):
            epilogue(utab, ugath, ulist, ublist, n_u, u_epi_w, u_main,
                     u_epi_main_w, u_full, u_tail_w, tailu, 1,
                     n_user - u_main, n_user)

        clear_subb()
        with jax.named_scope("imain"):
            main_blocks(itab, igath, ilist, iblist, n_i, i_nblk, 4, BLK_I)
        with jax.named_scope("iepi"):
            epilogue(itab, igath, ilist, iblist, n_i, i_epi_w, i_main,
                     i_epi_main_w, i_full, i_tail_w, taili, i_nsub,
                     i_sub_w, n_item)

    @functools.partial(
        pl.kernel,
        out_type=jax.ShapeDtypeStruct((BATCH,), jnp.float32),
        mesh=mesh,
        compiler_params=cparams,
        scratch_types=[
            pltpu.VMEM((256, 128), jnp.float32),
            pltpu.VMEM((256, 128), jnp.float32),
            pltpu.VMEM((b_per_w,), jnp.float32),
        ],
    )
    def dot_kernel(ugath, igath, out_hbm, ub, ib, outv):
        wid = lax.axis_index("s") * nc + lax.axis_index("c")
        lane = jax.lax.iota(jnp.int32, lanes)
        base = wid * b_per_w
        for p in range(b_per_w // 256):
            pltpu.sync_copy(ugath.at[pl.ds(base + p * 256, 256), :], ub)
            pltpu.sync_copy(igath.at[pl.ds(base + p * 256, 256), :], ib)

            def group(g, _, p=p):
                row = g * lanes + lane
                acc = jnp.zeros((lanes,), jnp.float32)
                for d in range(D):
                    dv = jnp.full((lanes,), d, jnp.int32)
                    u = plsc.load_gather(ub, [row, dv])
                    v = plsc.load_gather(ib, [row, dv])
                    acc = acc + u * v
                outv[pl.ds(p * 256 + g * lanes, lanes)] = acc
                return 0

            lax.fori_loop(0, 256 // lanes, group, 0)
        pltpu.sync_copy(outv, out_hbm.at[pl.ds(base, b_per_w)])

    return gather_kernel, dot_kernel


def kernel(user, item, user_table, item_table):
    k1, k2 = _build(user_table.shape[0], item_table.shape[0])
    ugath, igath = k1(user.astype(jnp.int32), item.astype(jnp.int32),
                      user_table.T, item_table.T)
    return k2(ugath, igath)


# speedup vs baseline: 1.1494x; 1.1494x over previous
import functools

import jax
import jax.numpy as jnp
from jax import lax
from jax.experimental import pallas as pl
from jax.experimental.pallas import tpu as pltpu
from jax.experimental.pallas import tpu_sc as plsc

EMBED_DIM = 64
BATCH = 16384
ROW_W = 128
IDX_CHUNK = 128
PASS_ROWS = 256


@functools.cache
def _build(num_users: int, num_items: int):
    info = plsc.get_sparse_core_info()
    nc, ns, lanes = info.num_cores, info.num_subcores, info.num_lanes
    nw = nc * ns
    b_per_w = BATCH // nw
    n_pass = b_per_w // PASS_ROWS
    n_chunks = PASS_ROWS // IDX_CHUNK
    n_groups = PASS_ROWS // lanes

    mesh = plsc.VectorSubcoreMesh(core_axis_name="c", subcore_axis_name="s")

    @functools.partial(
        pl.kernel,
        out_type=jax.ShapeDtypeStruct((BATCH,), jnp.float32),
        mesh=mesh,
        compiler_params=pltpu.CompilerParams(needs_layout_passes=False),
        scratch_types=[
            pltpu.VMEM((n_chunks, IDX_CHUNK), jnp.int32),
            pltpu.VMEM((n_chunks, IDX_CHUNK), jnp.int32),
            pltpu.VMEM((b_per_w,), jnp.int32),
            pltpu.VMEM((b_per_w,), jnp.int32),
            pltpu.VMEM((PASS_ROWS, ROW_W), jnp.float32),
            pltpu.VMEM((PASS_ROWS, ROW_W), jnp.float32),
            pltpu.VMEM((b_per_w,), jnp.float32),
            pltpu.SemaphoreType.DMA,
        ],
    )
    def sc_kernel(urow_hbm, irow_hbm, uoff_hbm, ioff_hbm,
                  utab_hbm, itab_hbm, out_hbm,
                  uidx, iidx, uoffv, ioffv, urows, irows, outv, sem):
        wid = lax.axis_index("s") * nc + lax.axis_index("c")
        base = wid * b_per_w

        pltpu.sync_copy(uoff_hbm.at[pl.ds(base, b_per_w)], uoffv)
        pltpu.sync_copy(ioff_hbm.at[pl.ds(base, b_per_w)], ioffv)

        lane = jax.lax.iota(jnp.int32, lanes)

        for p in range(n_pass):
            pbase = base + p * PASS_ROWS
            for j in range(n_chunks):
                pltpu.sync_copy(
                    urow_hbm.at[pl.ds(pbase + j * IDX_CHUNK, IDX_CHUNK)],
                    uidx.at[j])
                pltpu.sync_copy(
                    irow_hbm.at[pl.ds(pbase + j * IDX_CHUNK, IDX_CHUNK)],
                    iidx.at[j])

            copies = []
            for j in range(n_chunks):
                dst = pl.ds(j * IDX_CHUNK, IDX_CHUNK)
                copies.append(pltpu.async_copy(utab_hbm.at[uidx.at[j]],
                                               urows.at[dst], sem))
                copies.append(pltpu.async_copy(itab_hbm.at[iidx.at[j]],
                                               irows.at[dst], sem))
            for c in copies:
                c.wait()

            def group_body(g, _, p=p):
                row = g * lanes + lane
                ucol0 = uoffv[pl.ds(p * PASS_ROWS + g * lanes, lanes)]
                icol0 = ioffv[pl.ds(p * PASS_ROWS + g * lanes, lanes)]
                acc = jnp.zeros((lanes,), jnp.float32)
                for d in range(EMBED_DIM):
                    u = plsc.load_gather(urows, [row, ucol0 + d])
                    v = plsc.load_gather(irows, [row, icol0 + d])
                    acc = acc + u * v
                outv[pl.ds(p * PASS_ROWS + g * lanes, lanes)] = acc
                return 0

            lax.fori_loop(0, n_groups, group_body, 0)

        pltpu.sync_copy(outv, out_hbm.at[pl.ds(base, b_per_w)])

    return sc_kernel


def kernel(user, item, user_table, item_table):
    nu, nd = user_table.shape
    ni, _ = item_table.shape
    fn = _build(nu, ni)
    user = user.astype(jnp.int32)
    item = item.astype(jnp.int32)
    zero = jnp.zeros_like(user)
    return fn(
        user, item, zero, zero,
        jnp.pad(user_table, ((0, 0), (0, ROW_W - nd))),
        jnp.pad(item_table, ((0, 0), (0, ROW_W - nd))),
    )
